# baseline (speedup 1.0000x reference)
import functools

import jax
import jax.numpy as jnp
from jax.experimental import pallas as pl

_B, _F, _H, _W = 16, 8, 512, 512
_HB = 64


def _body(in_ref, tg_ref, mk_ref, out_ref):
    i = pl.program_id(0)

    @pl.when(i == 0)
    def _init():
        out_ref[...] = jnp.zeros_like(out_ref)

    a = jnp.abs(in_ref[0] - tg_ref[0])
    m = mk_ref[0].astype(jnp.float32)
    anym = jnp.max(m, axis=0, keepdims=True)

    def red(x):
        return jnp.sum(x.reshape(_F, _HB, _W // 128, 128), axis=(1, 2))

    out_ref[0] += red(a * m)
    out_ref[1] += red(m)
    out_ref[2] += red(a * (1.0 - m))
    out_ref[3] += red(a * anym)
    out_ref[4, 0] += jnp.sum(
        anym.reshape(_HB, _W // 128, 128), axis=(0, 1))


@jax.jit
def kernel(input, target, masks, hull):
    del hull
    grid = (_B * (_H // _HB),)

    def im(i):
        return (i // (_H // _HB), 0, i % (_H // _HB), 0)

    acc = pl.pallas_call(
        _body,
        grid=grid,
        in_specs=[
            pl.BlockSpec((1, _F, _HB, _W), im),
            pl.BlockSpec((1, _F, _HB, _W), im),
            pl.BlockSpec((1, _F, _HB, _W), im),
        ],
        out_specs=pl.BlockSpec((5, _F, 128), lambda i: (0, 0, 0)),
        out_shape=jax.ShapeDtypeStruct((5, _F, 128), jnp.float32),
    )(input, target, masks)

    sums = jnp.sum(acc, axis=-1)
    s_mask, c_mask, s_not, s_all = sums[0], sums[1], sums[2], sums[3]
    c_all = jnp.sum(acc[4, 0])
    c_not = float(_B * _H * _W) - c_mask

    def mmean(s, c):
        return jnp.where(c > 0, s / jnp.maximum(c, 1.0), jnp.zeros_like(s))

    lf = jnp.mean(mmean(s_mask, c_mask))
    lb = jnp.mean(mmean(s_not, c_not))
    la = jnp.mean(mmean(s_all, jnp.full_like(s_all, c_all)))
    return (lf + la + lb) / 3.0


# speedup vs baseline: 1.3417x; 1.3417x over previous
import functools

import jax
import jax.numpy as jnp
from jax.experimental import pallas as pl

_B, _F, _H, _W = 16, 8, 512, 512
_HB = 64


def _body(in_ref, tg_ref, mk_ref, out_ref):
    i = pl.program_id(0)

    @pl.when(i == 0)
    def _init():
        out_ref[...] = jnp.zeros_like(out_ref)

    a = jnp.abs(in_ref[0] - tg_ref[0])
    m = mk_ref[0].astype(jnp.float32)
    am = a * m
    anym = jnp.max(m, axis=0, keepdims=True)

    out_ref[0] += jnp.sum(am, axis=1)
    out_ref[1] += jnp.sum(m, axis=1)
    out_ref[2] += jnp.sum(a - am, axis=1)
    out_ref[3] += jnp.sum(a * anym, axis=1)
    out_ref[4, :1] += jnp.sum(anym, axis=1)


@jax.jit
def kernel(input, target, masks, hull):
    del hull
    grid = (_B * (_H // _HB),)

    def im(i):
        return (i // (_H // _HB), 0, i % (_H // _HB), 0)

    acc = pl.pallas_call(
        _body,
        grid=grid,
        in_specs=[
            pl.BlockSpec((1, _F, _HB, _W), im),
            pl.BlockSpec((1, _F, _HB, _W), im),
            pl.BlockSpec((1, _F, _HB, _W), im),
        ],
        out_specs=pl.BlockSpec((5, _F, _W), lambda i: (0, 0, 0)),
        out_shape=jax.ShapeDtypeStruct((5, _F, _W), jnp.float32),
    )(input, target, masks)

    sums = jnp.sum(acc, axis=-1)
    s_mask, c_mask, s_not, s_all = sums[0], sums[1], sums[2], sums[3]
    c_all = jnp.sum(acc[4, 0])
    c_not = float(_B * _H * _W) - c_mask

    def mmean(s, c):
        return jnp.where(c > 0, s / jnp.maximum(c, 1.0), jnp.zeros_like(s))

    lf = jnp.mean(mmean(s_mask, c_mask))
    lb = jnp.mean(mmean(s_not, c_not))
    la = jnp.mean(mmean(s_all, jnp.full_like(s_all, c_all)))
    return (lf + la + lb) / 3.0


# speedup vs baseline: 1.5495x; 1.1549x over previous
import functools

import jax
import jax.numpy as jnp
from jax.experimental import pallas as pl

_B, _F, _H, _W = 16, 8, 512, 512
_HB = 128


def _body(in_ref, tg_ref, mk_ref, out_ref):
    i = pl.program_id(0)

    @pl.when(i == 0)
    def _init():
        out_ref[...] = jnp.zeros_like(out_ref)

    a = jnp.abs(in_ref[0] - tg_ref[0])
    m = mk_ref[0].astype(jnp.float32)
    am = a * m
    anym = jnp.max(m, axis=0, keepdims=True)

    out_ref[0] += jnp.sum(am, axis=1)
    out_ref[1] += jnp.sum(m, axis=1)
    out_ref[2] += jnp.sum(a - am, axis=1)
    out_ref[3] += jnp.sum(a * anym, axis=1)
    out_ref[4, :1] += jnp.sum(anym, axis=1)


@jax.jit
def kernel(input, target, masks, hull):
    del hull
    grid = (_B * (_H // _HB),)

    def im(i):
        return (i // (_H // _HB), 0, i % (_H // _HB), 0)

    acc = pl.pallas_call(
        _body,
        grid=grid,
        in_specs=[
            pl.BlockSpec((1, _F, _HB, _W), im),
            pl.BlockSpec((1, _F, _HB, _W), im),
            pl.BlockSpec((1, _F, _HB, _W), im),
        ],
        out_specs=pl.BlockSpec((5, _F, _W), lambda i: (0, 0, 0)),
        out_shape=jax.ShapeDtypeStruct((5, _F, _W), jnp.float32),
    )(input, target, masks)

    sums = jnp.sum(acc, axis=-1)
    s_mask, c_mask, s_not, s_all = sums[0], sums[1], sums[2], sums[3]
    c_all = jnp.sum(acc[4, 0])
    c_not = float(_B * _H * _W) - c_mask

    def mmean(s, c):
        return jnp.where(c > 0, s / jnp.maximum(c, 1.0), jnp.zeros_like(s))

    lf = jnp.mean(mmean(s_mask, c_mask))
    lb = jnp.mean(mmean(s_not, c_not))
    la = jnp.mean(mmean(s_all, jnp.full_like(s_all, c_all)))
    return (lf + la + lb) / 3.0


# speedup vs baseline: 1.6575x; 1.0697x over previous
import functools

import jax
import jax.numpy as jnp
from jax.experimental import pallas as pl

_B, _F, _H, _W = 16, 8, 512, 512
_HB = 256


def _body(in_ref, tg_ref, mk_ref, out_ref):
    i = pl.program_id(0)

    @pl.when(i == 0)
    def _init():
        out_ref[...] = jnp.zeros_like(out_ref)

    a = jnp.abs(in_ref[0] - tg_ref[0])
    m = mk_ref[0].astype(jnp.float32)
    am = a * m
    anym = jnp.max(m, axis=0, keepdims=True)

    out_ref[0] += jnp.sum(am, axis=1)
    out_ref[1] += jnp.sum(m, axis=1)
    out_ref[2] += jnp.sum(a - am, axis=1)
    out_ref[3] += jnp.sum(a * anym, axis=1)
    out_ref[4, :1] += jnp.sum(anym, axis=1)


@jax.jit
def kernel(input, target, masks, hull):
    del hull
    grid = (_B * (_H // _HB),)

    def im(i):
        return (i // (_H // _HB), 0, i % (_H // _HB), 0)

    acc = pl.pallas_call(
        _body,
        grid=grid,
        in_specs=[
            pl.BlockSpec((1, _F, _HB, _W), im),
            pl.BlockSpec((1, _F, _HB, _W), im),
            pl.BlockSpec((1, _F, _HB, _W), im),
        ],
        out_specs=pl.BlockSpec((5, _F, _W), lambda i: (0, 0, 0)),
        out_shape=jax.ShapeDtypeStruct((5, _F, _W), jnp.float32),
    )(input, target, masks)

    sums = jnp.sum(acc, axis=-1)
    s_mask, c_mask, s_not, s_all = sums[0], sums[1], sums[2], sums[3]
    c_all = jnp.sum(acc[4, 0])
    c_not = float(_B * _H * _W) - c_mask

    def mmean(s, c):
        return jnp.where(c > 0, s / jnp.maximum(c, 1.0), jnp.zeros_like(s))

    lf = jnp.mean(mmean(s_mask, c_mask))
    lb = jnp.mean(mmean(s_not, c_not))
    la = jnp.mean(mmean(s_all, jnp.full_like(s_all, c_all)))
    return (lf + la + lb) / 3.0


# speedup vs baseline: 1.8939x; 1.1426x over previous
import jax
import jax.numpy as jnp
from jax.experimental import pallas as pl

_B, _F, _H, _W = 16, 8, 512, 512
_HB = 256


def _body(in_ref, tg_ref, mb_ref, out_ref):
    i = pl.program_id(0)

    @pl.when(i == 0)
    def _init():
        out_ref[...] = jnp.zeros_like(out_ref)

    mi = mb_ref[0].astype(jnp.int32)
    shifts = jax.lax.broadcasted_iota(jnp.int32, (_F, 1, 1), 0)
    m = ((mi[None] >> shifts) & 1).astype(jnp.float32)
    anym = (mi[None] != 0).astype(jnp.float32)

    a = jnp.abs(in_ref[0] - tg_ref[0])
    am = a * m

    out_ref[0] += jnp.sum(am, axis=1)
    out_ref[1] += jnp.sum(m, axis=1)
    out_ref[2] += jnp.sum(a - am, axis=1)
    out_ref[3] += jnp.sum(a * anym, axis=1)
    out_ref[4, :1] += jnp.sum(anym, axis=1)


@jax.jit
def kernel(input, target, masks, hull):
    del hull
    weights = (1 << jnp.arange(_F, dtype=jnp.int32)).reshape(1, _F, 1, 1)
    mbits = jnp.sum(masks * weights, axis=1).astype(jnp.uint8)

    grid = (_B * (_H // _HB),)
    nh = _H // _HB

    def im4(i):
        return (i // nh, 0, i % nh, 0)

    def im3(i):
        return (i // nh, i % nh, 0)

    acc = pl.pallas_call(
        _body,
        grid=grid,
        in_specs=[
            pl.BlockSpec((1, _F, _HB, _W), im4),
            pl.BlockSpec((1, _F, _HB, _W), im4),
            pl.BlockSpec((1, _HB, _W), im3),
        ],
        out_specs=pl.BlockSpec((5, _F, _W), lambda i: (0, 0, 0)),
        out_shape=jax.ShapeDtypeStruct((5, _F, _W), jnp.float32),
    )(input, target, mbits)

    sums = jnp.sum(acc, axis=-1)
    s_mask, c_mask, s_not, s_all = sums[0], sums[1], sums[2], sums[3]
    c_all = jnp.sum(acc[4, 0])
    c_not = float(_B * _H * _W) - c_mask

    def mmean(s, c):
        return jnp.where(c > 0, s / jnp.maximum(c, 1.0), jnp.zeros_like(s))

    lf = jnp.mean(mmean(s_mask, c_mask))
    lb = jnp.mean(mmean(s_not, c_not))
    la = jnp.mean(mmean(s_all, jnp.full_like(s_all, c_all)))
    return (lf + la + lb) / 3.0
